# baseline (speedup 1.0000x reference)
import jax
import jax.numpy as jnp
from jax.experimental import pallas as pl
from jax.experimental.pallas import tpu as pltpu

_NUM_CLUSTERS = 1024
_D = 128
_BM = 1024


def _mc_kernel(e_ref, c_ref, o_ref, cn_ref):
    @pl.when(pl.program_id(0) == 0)
    def _():
        c = c_ref[...]
        cs = jnp.sum(c * c, axis=-1, keepdims=True)
        cn_ref[...] = c * jax.lax.rsqrt(jnp.maximum(cs, 1e-24))

    e = e_ref[...]
    scale = 14.426950408889634
    es = jnp.sum(e * e, axis=-1, keepdims=True)
    en = e * (scale * jax.lax.rsqrt(jnp.maximum(es, 1e-24)))
    sim = jax.lax.dot_general(
        en, cn_ref[...], (((1,), (1,)), ((), ())), preferred_element_type=jnp.float32
    )
    p = jnp.exp2(sim)
    o_ref[...] = p * (1.0 / jnp.sum(p, axis=-1, keepdims=True))


def kernel(embeddings, centroids, importance):
    del importance
    batch = embeddings.shape[0]
    return pl.pallas_call(
        _mc_kernel,
        grid=(batch // _BM,),
        in_specs=[
            pl.BlockSpec((_BM, _D), lambda i: (i, 0)),
            pl.BlockSpec((_NUM_CLUSTERS, _D), lambda i: (0, 0)),
        ],
        out_specs=pl.BlockSpec((_BM, _NUM_CLUSTERS), lambda i: (i, 0)),
        out_shape=jax.ShapeDtypeStruct((batch, _NUM_CLUSTERS), jnp.float32),
        scratch_shapes=[pltpu.VMEM((_NUM_CLUSTERS, _D), jnp.float32)],
        compiler_params=pltpu.CompilerParams(dimension_semantics=("arbitrary",)),
    )(embeddings, centroids)


# speedup vs baseline: 1.1391x; 1.1391x over previous
import jax
import jax.numpy as jnp
from jax.experimental import pallas as pl
from jax.experimental.pallas import tpu as pltpu

_NUM_CLUSTERS = 1024
_D = 128
_BM = 2048


def _mc_kernel(e_ref, c_ref, o_ref, cn_ref):
    @pl.when(pl.program_id(0) == 0)
    def _():
        c = c_ref[...]
        cs = jnp.sum(c * c, axis=-1, keepdims=True)
        cn_ref[...] = c * jax.lax.rsqrt(jnp.maximum(cs, 1e-24))

    e = e_ref[...]
    scale = 14.426950408889634
    es = jnp.sum(e * e, axis=-1, keepdims=True)
    en = e * (scale * jax.lax.rsqrt(jnp.maximum(es, 1e-24)))
    sim = jax.lax.dot_general(
        en, cn_ref[...], (((1,), (1,)), ((), ())), preferred_element_type=jnp.float32
    )
    p = jnp.exp2(sim)
    o_ref[...] = p * (1.0 / jnp.sum(p, axis=-1, keepdims=True))


def kernel(embeddings, centroids, importance):
    del importance
    batch = embeddings.shape[0]
    return pl.pallas_call(
        _mc_kernel,
        grid=(batch // _BM,),
        in_specs=[
            pl.BlockSpec((_BM, _D), lambda i: (i, 0)),
            pl.BlockSpec((_NUM_CLUSTERS, _D), lambda i: (0, 0)),
        ],
        out_specs=pl.BlockSpec((_BM, _NUM_CLUSTERS), lambda i: (i, 0)),
        out_shape=jax.ShapeDtypeStruct((batch, _NUM_CLUSTERS), jnp.float32),
        scratch_shapes=[pltpu.VMEM((_NUM_CLUSTERS, _D), jnp.float32)],
        compiler_params=pltpu.CompilerParams(dimension_semantics=("arbitrary",)),
    )(embeddings, centroids)
